# Initial kernel scaffold; baseline (speedup 1.0000x reference)
import jax
import jax.numpy as jnp
from jax.experimental import pallas as pl


def kernel(x, W_enc, b_enc, W_dec, b_dec):
    raise NotImplementedError("write your pallas kernel here")


# speedup vs baseline: 11.7949x; 11.7949x over previous
import functools

import jax
import jax.numpy as jnp
from jax.experimental import pallas as pl
from jax.experimental.pallas import tpu as pltpu

K_TOP = 32
G = 256


def _body(xb_ref, We_ref, be_ref, Wd_ref, bd_ref, h_ref, xhat_ref,
          pre_ref, acc_ref, a_ref, b3_ref, c3_ref, d4_ref, st_ref, thrf_ref,
          *, NJ, HJ, R):
    j = pl.program_id(1)

    def count_ge(t_bits):
        tot = jnp.zeros((R, 1), jnp.int32)
        for c in range(NJ):
            bits = jax.lax.bitcast_convert_type(
                pre_ref[:, c * HJ:(c + 1) * HJ], jnp.int32)
            tot = tot + jnp.sum((bits >= t_bits).astype(jnp.int32),
                                axis=1, keepdims=True)
        return tot

    @pl.when(j == 0)
    def _init():
        a_ref[...] = jnp.full((R, G), -1.0, jnp.float32)
        b3_ref[...] = jnp.full((R, G), -1.0, jnp.float32)
        c3_ref[...] = jnp.full((R, G), -1.0, jnp.float32)
        d4_ref[...] = jnp.full((R, G), -1.0, jnp.float32)

    @pl.when(j < NJ)
    def _encode():
        acc = jax.lax.dot_general(
            xb_ref[...], We_ref[...], (((1,), (1,)), ((), ())),
            preferred_element_type=jnp.float32)
        pre = jnp.maximum(acc + be_ref[...], 0.0)
        pre_ref[:, pl.ds(j * HJ, HJ)] = pre
        A = a_ref[...]
        B = b3_ref[...]
        C = c3_ref[...]
        D4 = d4_ref[...]
        for s in range(HJ // G):
            v = pre[:, s * G:(s + 1) * G]
            nA = jnp.maximum(A, v)
            r = jnp.minimum(A, v)
            nB = jnp.maximum(B, r)
            r = jnp.minimum(B, r)
            nC = jnp.maximum(C, r)
            r = jnp.minimum(C, r)
            D4 = jnp.maximum(D4, r)
            A, B, C = nA, nB, nC
        a_ref[...] = A
        b3_ref[...] = B
        c3_ref[...] = C
        d4_ref[...] = D4

    @pl.when(j == NJ)
    def _threshold():
        rmax = jnp.max(a_ref[...], axis=1, keepdims=True)
        S = jnp.concatenate([a_ref[...], b3_ref[...], c3_ref[...], d4_ref[...]], axis=1)

        def ext(_, Sc):
            rm = jnp.max(Sc, axis=1, keepdims=True)
            return jnp.where(Sc == rm, -1.0, Sc)

        Sf = jax.lax.fori_loop(0, K_TOP - 1, ext, S)
        t32 = jnp.max(Sf, axis=1, keepdims=True)

        lo = jax.lax.bitcast_convert_type(t32, jnp.int32)
        hi = jax.lax.bitcast_convert_type(rmax, jnp.int32) + 1
        cnt = count_ge(lo)
        done = (cnt == K_TOP).astype(jnp.int32)
        st_ref[:, 0:1] = lo
        st_ref[:, 1:2] = hi
        st_ref[:, 2:3] = done
        st_ref[:, 3:4] = lo

        def fb(_i, carry):
            undone = jnp.sum(1 - st_ref[:, 2:3])

            @pl.when(undone > 0)
            def _step():
                flo = st_ref[:, 0:1]
                fhi = st_ref[:, 1:2]
                fdone = st_ref[:, 2:3]
                fthr = st_ref[:, 3:4]
                mid = flo + (fhi - flo) // 2
                c2 = count_ge(mid)
                ge = c2 >= K_TOP
                hit = (c2 == K_TOP) & (fdone == 0)
                nlo = jnp.where(ge, mid, flo)
                nhi = jnp.where(ge, fhi, mid)
                narrow = (nhi - nlo) <= 1
                act = fdone == 0
                st_ref[:, 0:1] = jnp.where(act, nlo, flo)
                st_ref[:, 1:2] = jnp.where(act, nhi, fhi)
                st_ref[:, 3:4] = jnp.where(
                    hit, mid, jnp.where(act & narrow, nlo, fthr))
                st_ref[:, 2:3] = jnp.where(
                    act & (hit | narrow), 1, fdone)
            return carry

        jax.lax.fori_loop(0, 31, fb, 0)
        thrf_ref[...] = jax.lax.bitcast_convert_type(
            st_ref[:, 3:4], jnp.float32)
        acc_ref[...] = jnp.zeros_like(acc_ref)

    @pl.when(j >= NJ)
    def _mask_decode():
        pre_blk = pre_ref[:, pl.ds((j - NJ) * HJ, HJ)]
        hblk = jnp.where(pre_blk >= thrf_ref[...], pre_blk, 0.0)
        h_ref[...] = hblk
        acc_ref[...] += jax.lax.dot_general(
            hblk.astype(jnp.bfloat16), Wd_ref[...], (((1,), (1,)), ((), ())),
            preferred_element_type=jnp.float32)

        @pl.when(j == 2 * NJ - 1)
        def _finish():
            xhat_ref[...] = acc_ref[...] + bd_ref[...]


@jax.jit
def kernel(x, W_enc, b_enc, W_dec, b_dec):
    B, D = x.shape
    H = W_enc.shape[0]
    R = min(512, B)
    NI = B // R
    HJ = min(512, H)
    NJ = H // HJ

    xb = x.astype(jnp.bfloat16)
    We = W_enc.astype(jnp.bfloat16)
    Wd = W_dec.astype(jnp.bfloat16)
    be2 = b_enc.reshape(1, H)
    bd2 = b_dec.reshape(1, D)

    grid = (NI, 2 * NJ)
    h, x_hat = pl.pallas_call(
        functools.partial(_body, NJ=NJ, HJ=HJ, R=R),
        grid=grid,
        in_specs=[
            pl.BlockSpec((R, D), lambda i, j: (i, 0)),
            pl.BlockSpec((HJ, D), lambda i, j: (jnp.minimum(j, NJ - 1), 0)),
            pl.BlockSpec((1, HJ), lambda i, j: (0, jnp.minimum(j, NJ - 1))),
            pl.BlockSpec((D, HJ), lambda i, j: (0, jnp.clip(j - NJ, 0, NJ - 1))),
            pl.BlockSpec((1, D), lambda i, j: (0, 0)),
        ],
        out_specs=[
            pl.BlockSpec((R, HJ), lambda i, j: (i, jnp.clip(j - NJ, 0, NJ - 1))),
            pl.BlockSpec((R, D), lambda i, j: (i, 0)),
        ],
        out_shape=[
            jax.ShapeDtypeStruct((B, H), jnp.float32),
            jax.ShapeDtypeStruct((B, D), jnp.float32),
        ],
        scratch_shapes=[
            pltpu.VMEM((R, H), jnp.float32),
            pltpu.VMEM((R, D), jnp.float32),
            pltpu.VMEM((R, G), jnp.float32),
            pltpu.VMEM((R, G), jnp.float32),
            pltpu.VMEM((R, G), jnp.float32),
            pltpu.VMEM((R, G), jnp.float32),
            pltpu.VMEM((R, 8), jnp.int32),
            pltpu.VMEM((R, 1), jnp.float32),
        ],
        compiler_params=pltpu.CompilerParams(
            dimension_semantics=("arbitrary", "arbitrary")),
    )(xb, We, be2, Wd, bd2)
    return (h, x_hat)
